# baseline (speedup 1.0000x reference)
import functools

import jax
import jax.numpy as jnp
from jax import lax
from jax.experimental import pallas as pl
from jax.experimental.pallas import tpu as pltpu
from jax.experimental.pallas import tpu_sc as plsc

N = 10000
E = 320000
D = 128
L = 16
NSC = 2
NTILE = 16
NW = NSC * NTILE
NPAD = 10240
CHUNK = 128
CPT = 80
EPAD = NW * CPT * CHUNK
NACC = NPAD
APT = NACC // NTILE
ZCP = APT // CHUNK
EDT = E // NW

_mesh = plsc.VectorSubcoreMesh(core_axis_name="c", subcore_axis_name="s",
                               num_cores=NSC, num_subcores=NTILE)


@functools.partial(
    pl.kernel,
    out_type=jax.ShapeDtypeStruct((NW, NPAD), jnp.float32),
    mesh=_mesh,
    scratch_types=[
        pltpu.VMEM((EDT,), jnp.int32),
        pltpu.VMEM((NPAD,), jnp.float32),
    ],
    compiler_params=pltpu.CompilerParams(needs_layout_passes=False),
)
def _deg_kernel(dst_hbm, out_hbm, dst_v, hist_v):
    c = lax.axis_index("c")
    s = lax.axis_index("s")
    wid = c * NTILE + s
    zeros16 = jnp.zeros((L,), jnp.float32)

    def zbody(i, _):
        hist_v[pl.ds(i * L, L)] = zeros16
        return ()

    lax.fori_loop(0, NPAD // L, zbody, (), unroll=8)
    pltpu.sync_copy(dst_hbm.at[pl.ds(wid * EDT, EDT)], dst_v)
    ones16 = jnp.ones((L,), jnp.float32)

    def body(k, _):
        idx = dst_v[pl.ds(k * L, L)]
        plsc.addupdate_scatter(hist_v, [idx], ones16)
        return ()

    lax.fori_loop(0, EDT // L, body, (), unroll=8)
    pltpu.sync_copy(hist_v, out_hbm.at[wid])


_BM = 1024


@functools.partial(
    pl.pallas_call,
    grid=(NPAD // _BM,),
    in_specs=[
        pl.BlockSpec((_BM, D), lambda i: (i, 0)),
        pl.BlockSpec((D, D), lambda i: (0, 0)),
        pl.BlockSpec((NW, _BM), lambda i: (0, i)),
    ],
    out_specs=pl.BlockSpec((_BM, D), lambda i: (i, 0)),
    out_shape=jax.ShapeDtypeStruct((NPAD, D), jnp.float32),
)
def _xw_kernel(x_ref, w_ref, h_ref, o_ref):
    deg = jnp.sum(h_ref[...], axis=0) + 1.0
    dis = lax.rsqrt(deg)
    xw = jnp.dot(x_ref[...], w_ref[...], preferred_element_type=jnp.float32)
    o_ref[...] = xw * dis[:, None]


@functools.partial(
    pl.kernel,
    out_type=jax.ShapeDtypeStruct((NSC, NACC, D), jnp.float32),
    mesh=_mesh,
    scratch_types=[
        pltpu.VMEM((CHUNK,), jnp.int32),
        pltpu.VMEM((CHUNK,), jnp.int32),
        pltpu.VMEM((CHUNK, D), jnp.float32),
        pltpu.VMEM_SHARED((NACC, D), jnp.float32),
        pltpu.SemaphoreType.DMA,
    ],
)
def _edge_kernel(xw_hbm, src_hbm, dst_hbm, out_hbm, src_v, dst_v, rows_v,
                 acc_sh, sem):
    c = lax.axis_index("c")
    s = lax.axis_index("s")
    base = (c * NTILE + s) * CPT * CHUNK
    zeros16 = jnp.zeros((L,), jnp.float32)

    def zrow(i, _):
        for j in range(D // L):
            rows_v[i, pl.ds(j * L, L)] = zeros16
        return ()

    lax.fori_loop(0, CHUNK, zrow, ())
    for i in range(ZCP):
        pltpu.sync_copy(rows_v, acc_sh.at[pl.ds(s * APT + i * CHUNK, CHUNK)])
    plsc.subcore_barrier()

    def body(g, _):
        off = base + g * CHUNK
        pltpu.sync_copy(src_hbm.at[pl.ds(off, CHUNK)], src_v)
        pltpu.sync_copy(dst_hbm.at[pl.ds(off, CHUNK)], dst_v)
        pltpu.async_copy(xw_hbm.at[src_v], rows_v, sem).wait()
        pltpu.sync_copy(rows_v, acc_sh.at[dst_v], add=True)
        return ()

    lax.fori_loop(0, CPT, body, ())
    plsc.subcore_barrier()

    for i in range(ZCP):
        r0 = s * APT + i * CHUNK
        pltpu.sync_copy(acc_sh.at[pl.ds(r0, CHUNK)],
                        out_hbm.at[c, pl.ds(r0, CHUNK)])


_BD = 1024


@functools.partial(
    pl.pallas_call,
    grid=(pl.cdiv(N, _BD),),
    in_specs=[
        pl.BlockSpec((NSC, _BD, D), lambda i: (0, i, 0)),
        pl.BlockSpec((_BD, D), lambda i: (i, 0)),
        pl.BlockSpec((NW, _BD), lambda i: (0, i)),
        pl.BlockSpec((1, D), lambda i: (0, 0)),
        pl.BlockSpec((1, D), lambda i: (0, 0)),
    ],
    out_specs=pl.BlockSpec((_BD, D), lambda i: (i, 0)),
    out_shape=jax.ShapeDtypeStruct((N, D), jnp.float32),
)
def _finish_kernel(acc_ref, xw_ref, h_ref, b_ref, a_ref, o_ref):
    deg = jnp.sum(h_ref[...], axis=0) + 1.0
    dis = lax.rsqrt(deg)
    acc = acc_ref[...]
    t = (acc[0] + acc[1] + xw_ref[...]) * dis[:, None] + b_ref[...]
    o_ref[...] = jnp.where(t >= 0, t, a_ref[...] * t)


def kernel(x, edge_index, W, b, prelu_a):
    src = edge_index[0]
    dst = edge_index[1]
    src_p = jnp.concatenate(
        [src, jnp.full((EPAD - E,), N, dtype=jnp.int32)]
    )
    dst_p = jnp.concatenate(
        [dst, jnp.zeros((EPAD - E,), dtype=jnp.int32)]
    )
    x_p = jnp.zeros((NPAD, D), x.dtype).at[:N].set(x)
    hist = _deg_kernel(dst)
    xw_s = _xw_kernel(x_p, W, hist)
    acc = _edge_kernel(xw_s, src_p, dst_p)
    out = _finish_kernel(acc, xw_s, hist, b.reshape(1, D),
                         prelu_a.reshape(1, D))
    return out


# speedup vs baseline: 1.3880x; 1.3880x over previous
import functools

import jax
import jax.numpy as jnp
from jax import lax
from jax.experimental import pallas as pl
from jax.experimental.pallas import tpu as pltpu
from jax.experimental.pallas import tpu_sc as plsc

N = 10000
E = 320000
D = 128
L = 16
NSC = 2
NTILE = 16
NW = NSC * NTILE
NPAD = 10240
CHUNK = 128
CPT = 79
EPT = CPT * CHUNK
EPAD = NW * CPT * CHUNK
NACC = NPAD
APT = NACC // NTILE
ZCP = APT // CHUNK

_mesh = plsc.VectorSubcoreMesh(core_axis_name="c", subcore_axis_name="s",
                               num_cores=NSC, num_subcores=NTILE)


@functools.partial(
    pl.kernel,
    out_type=jax.ShapeDtypeStruct((NW, NPAD), jnp.float32),
    mesh=_mesh,
    scratch_types=[
        pltpu.VMEM((EPT,), jnp.int32),
        pltpu.VMEM((NPAD,), jnp.float32),
    ],
    compiler_params=pltpu.CompilerParams(needs_layout_passes=False),
)
def _deg_kernel(dst_hbm, out_hbm, dst_v, hist_v):
    c = lax.axis_index("c")
    s = lax.axis_index("s")
    wid = c * NTILE + s
    zeros16 = jnp.zeros((L,), jnp.float32)

    def zbody(i, _):
        hist_v[pl.ds(i * L, L)] = zeros16
        return ()

    lax.fori_loop(0, NPAD // L, zbody, (), unroll=8)
    pltpu.sync_copy(dst_hbm.at[pl.ds(wid * EPT, EPT)], dst_v)
    ones16 = jnp.ones((L,), jnp.float32)

    def body(k, _):
        idx = dst_v[pl.ds(k * L, L)]
        plsc.addupdate_scatter(hist_v, [idx], ones16)
        return ()

    lax.fori_loop(0, EPT // L, body, (), unroll=8)
    pltpu.sync_copy(hist_v, out_hbm.at[wid])


_BM = 1024


@functools.partial(
    pl.pallas_call,
    grid=(NPAD // _BM,),
    in_specs=[
        pl.BlockSpec((_BM, D), lambda i: (i, 0)),
        pl.BlockSpec((D, D), lambda i: (0, 0)),
        pl.BlockSpec((NW, _BM), lambda i: (0, i)),
    ],
    out_specs=pl.BlockSpec((_BM, D), lambda i: (i, 0)),
    out_shape=jax.ShapeDtypeStruct((NPAD, D), jnp.float32),
)
def _xw_kernel(x_ref, w_ref, h_ref, o_ref):
    deg = jnp.sum(h_ref[...], axis=0) + 1.0
    dis = lax.rsqrt(deg)
    xw = jnp.dot(x_ref[...], w_ref[...], preferred_element_type=jnp.float32)
    o_ref[...] = xw * dis[:, None]


@functools.partial(
    pl.kernel,
    out_type=jax.ShapeDtypeStruct((NSC, NACC, D), jnp.float32),
    mesh=_mesh,
    scratch_types=[
        pltpu.VMEM((CHUNK,), jnp.int32),
        pltpu.VMEM((CHUNK,), jnp.int32),
        pltpu.VMEM((CHUNK, D), jnp.float32),
        pltpu.VMEM_SHARED((NACC, D), jnp.float32),
        pltpu.SemaphoreType.DMA,
    ],
)
def _edge_kernel(xw_hbm, src_hbm, dst_hbm, out_hbm, src_v, dst_v, rows_v,
                 acc_sh, sem):
    c = lax.axis_index("c")
    s = lax.axis_index("s")
    base = (c * NTILE + s) * CPT * CHUNK
    zeros16 = jnp.zeros((L,), jnp.float32)

    def zrow(i, _):
        for j in range(D // L):
            rows_v[i, pl.ds(j * L, L)] = zeros16
        return ()

    lax.fori_loop(0, CHUNK, zrow, ())
    for i in range(ZCP):
        pltpu.sync_copy(rows_v, acc_sh.at[pl.ds(s * APT + i * CHUNK, CHUNK)])
    plsc.subcore_barrier()

    def body(g, _):
        off = base + g * CHUNK
        pltpu.sync_copy(src_hbm.at[pl.ds(off, CHUNK)], src_v)
        pltpu.sync_copy(dst_hbm.at[pl.ds(off, CHUNK)], dst_v)
        pltpu.async_copy(xw_hbm.at[src_v], rows_v, sem).wait()
        pltpu.sync_copy(rows_v, acc_sh.at[dst_v], add=True)
        return ()

    lax.fori_loop(0, CPT, body, ())
    plsc.subcore_barrier()

    for i in range(ZCP):
        r0 = s * APT + i * CHUNK
        pltpu.sync_copy(acc_sh.at[pl.ds(r0, CHUNK)],
                        out_hbm.at[c, pl.ds(r0, CHUNK)])


_BD = 1024


@functools.partial(
    pl.pallas_call,
    grid=(pl.cdiv(N, _BD),),
    in_specs=[
        pl.BlockSpec((NSC, _BD, D), lambda i: (0, i, 0)),
        pl.BlockSpec((_BD, D), lambda i: (i, 0)),
        pl.BlockSpec((NW, _BD), lambda i: (0, i)),
        pl.BlockSpec((1, D), lambda i: (0, 0)),
        pl.BlockSpec((1, D), lambda i: (0, 0)),
    ],
    out_specs=pl.BlockSpec((_BD, D), lambda i: (i, 0)),
    out_shape=jax.ShapeDtypeStruct((N, D), jnp.float32),
)
def _finish_kernel(acc_ref, xw_ref, h_ref, b_ref, a_ref, o_ref):
    deg = jnp.sum(h_ref[...], axis=0) + 1.0
    dis = lax.rsqrt(deg)
    acc = acc_ref[...]
    t = (acc[0] + acc[1] + xw_ref[...]) * dis[:, None] + b_ref[...]
    o_ref[...] = jnp.where(t >= 0, t, a_ref[...] * t)


def kernel(x, edge_index, W, b, prelu_a):
    src = edge_index[0]
    dst = edge_index[1]
    pad = jnp.full((EPAD - E,), N, dtype=jnp.int32)
    src_p = jnp.concatenate([src, pad])
    dst_p = jnp.concatenate([dst, pad])
    x_p = jnp.zeros((NPAD, D), x.dtype).at[:N].set(x)
    hist = _deg_kernel(dst_p)
    xw_s = _xw_kernel(x_p, W, hist)
    acc = _edge_kernel(xw_s, src_p, dst_p)
    out = _finish_kernel(acc, xw_s, hist, b.reshape(1, D),
                         prelu_a.reshape(1, D))
    return out


# speedup vs baseline: 1.4055x; 1.0126x over previous
import functools

import jax
import jax.numpy as jnp
from jax import lax
from jax.experimental import pallas as pl
from jax.experimental.pallas import tpu as pltpu
from jax.experimental.pallas import tpu_sc as plsc

N = 10000
E = 320000
D = 128
L = 16
NSC = 2
NTILE = 16
NW = NSC * NTILE
NPAD = 10240
CHUNK = 128
CPT = 80
EPT = CPT * CHUNK
EPAD = NW * CPT * CHUNK
NACC = NPAD
APT = NACC // NTILE
ZCP = APT // CHUNK
NBUF = 2
NIDX = 4

_mesh = plsc.VectorSubcoreMesh(core_axis_name="c", subcore_axis_name="s",
                               num_cores=NSC, num_subcores=NTILE)


@functools.partial(
    pl.kernel,
    out_type=jax.ShapeDtypeStruct((NW, NPAD), jnp.float32),
    mesh=_mesh,
    scratch_types=[
        pltpu.VMEM((EPT,), jnp.int32),
        pltpu.VMEM((NPAD,), jnp.float32),
    ],
    compiler_params=pltpu.CompilerParams(needs_layout_passes=False),
)
def _deg_kernel(dst_hbm, out_hbm, dst_v, hist_v):
    c = lax.axis_index("c")
    s = lax.axis_index("s")
    wid = c * NTILE + s
    zeros16 = jnp.zeros((L,), jnp.float32)

    def zbody(i, _):
        hist_v[pl.ds(i * L, L)] = zeros16
        return ()

    lax.fori_loop(0, NPAD // L, zbody, (), unroll=8)
    pltpu.sync_copy(dst_hbm.at[pl.ds(wid * EPT, EPT)], dst_v)
    ones16 = jnp.ones((L,), jnp.float32)

    def body(k, _):
        idx = dst_v[pl.ds(k * L, L)]
        plsc.addupdate_scatter(hist_v, [idx], ones16)
        return ()

    lax.fori_loop(0, EPT // L, body, (), unroll=8)
    pltpu.sync_copy(hist_v, out_hbm.at[wid])


_BM = 1024


@functools.partial(
    pl.pallas_call,
    grid=(NPAD // _BM,),
    in_specs=[
        pl.BlockSpec((_BM, D), lambda i: (i, 0)),
        pl.BlockSpec((D, D), lambda i: (0, 0)),
        pl.BlockSpec((NW, _BM), lambda i: (0, i)),
    ],
    out_specs=pl.BlockSpec((_BM, D), lambda i: (i, 0)),
    out_shape=jax.ShapeDtypeStruct((NPAD, D), jnp.float32),
)
def _xw_kernel(x_ref, w_ref, h_ref, o_ref):
    deg = jnp.sum(h_ref[...], axis=0) + 1.0
    dis = lax.rsqrt(deg)
    xw = jnp.dot(x_ref[...], w_ref[...], preferred_element_type=jnp.float32)
    o_ref[...] = xw * dis[:, None]


@functools.partial(
    pl.kernel,
    out_type=jax.ShapeDtypeStruct((NSC, NACC, D), jnp.float32),
    mesh=_mesh,
    scratch_types=[
        [pltpu.VMEM((CHUNK,), jnp.int32)] * NIDX,
        [pltpu.VMEM((CHUNK,), jnp.int32)] * NIDX,
        pltpu.VMEM((NBUF, CHUNK, D), jnp.float32),
        pltpu.VMEM_SHARED((NACC, D), jnp.float32),
    ] + [pltpu.SemaphoreType.DMA] * (2 * NBUF + 2 * NIDX),
)
def _edge_kernel(xw_hbm, src_hbm, dst_hbm, out_hbm, src_v, dst_v, rows_v,
                 acc_sh, *sems):
    sg = sems[:NBUF]
    ss = sems[NBUF:2 * NBUF]
    ssrc = sems[2 * NBUF:2 * NBUF + NIDX]
    sdst = sems[2 * NBUF + NIDX:]
    c = lax.axis_index("c")
    s = lax.axis_index("s")
    base = (c * NTILE + s) * CPT * CHUNK
    zeros16 = jnp.zeros((L,), jnp.float32)

    def zrow(i, _):
        for j in range(D // L):
            rows_v[0, i, pl.ds(j * L, L)] = zeros16
        return ()

    lax.fori_loop(0, CHUNK, zrow, ())
    for i in range(ZCP):
        pltpu.sync_copy(rows_v.at[0],
                        acc_sh.at[pl.ds(s * APT + i * CHUNK, CHUNK)])
    plsc.subcore_barrier()

    def start_idx(g, q):
        off = base + g * CHUNK
        pltpu.async_copy(src_hbm.at[pl.ds(off, CHUNK)], src_v[q], ssrc[q])
        pltpu.async_copy(dst_hbm.at[pl.ds(off, CHUNK)], dst_v[q], sdst[q])

    def wait_idx(g, q):
        off = base + g * CHUNK
        pltpu.make_async_copy(src_hbm.at[pl.ds(off, CHUNK)], src_v[q],
                              ssrc[q]).wait()
        pltpu.make_async_copy(dst_hbm.at[pl.ds(off, CHUNK)], dst_v[q],
                              sdst[q]).wait()

    def start_gather(q, b):
        pltpu.async_copy(xw_hbm.at[src_v[q]], rows_v.at[b], sg[b])

    def wait_gather(q, b):
        pltpu.make_async_copy(xw_hbm.at[src_v[q]], rows_v.at[b],
                              sg[b]).wait()

    def start_scatter(q, b):
        pltpu.async_copy(rows_v.at[b], acc_sh.at[dst_v[q]], ss[b],
                         add=True)

    def wait_scatter(q, b):
        pltpu.make_async_copy(rows_v.at[b], acc_sh.at[dst_v[q]],
                              ss[b]).wait()

    def body(g, b, q, first, issue_gather, issue_idx):
        wait_gather(q, b)
        if not first:
            wait_scatter((q - 1) % NIDX, 1 - b)
        start_scatter(q, b)
        if issue_gather:
            wait_idx(g + 1, (q + 1) % NIDX)
            start_gather((q + 1) % NIDX, 1 - b)
        if issue_idx:
            start_idx(g + 3, (q + 3) % NIDX)

    for q in range(NIDX - 1):
        start_idx(q, q)
    wait_idx(0, 0)
    start_gather(0, 0)
    body(0, 0, 0, True, True, True)

    def quad(k, _):
        for j in range(NIDX):
            body(1 + 4 * k + j, (1 + j) % 2, (1 + j) % NIDX,
                 False, True, True)
        return ()

    lax.fori_loop(0, (CPT - 4) // NIDX, quad, ())
    body(CPT - 3, (CPT - 3) % 2, (CPT - 3) % NIDX, False, True, False)
    body(CPT - 2, (CPT - 2) % 2, (CPT - 2) % NIDX, False, True, False)
    g_last = CPT - 1
    body(g_last, g_last % 2, g_last % NIDX, False, False, False)
    wait_scatter(g_last % NIDX, g_last % 2)
    plsc.subcore_barrier()

    for i in range(ZCP):
        r0 = s * APT + i * CHUNK
        pltpu.sync_copy(acc_sh.at[pl.ds(r0, CHUNK)],
                        out_hbm.at[c, pl.ds(r0, CHUNK)])


_BD = 1024


@functools.partial(
    pl.pallas_call,
    grid=(pl.cdiv(N, _BD),),
    in_specs=[
        pl.BlockSpec((NSC, _BD, D), lambda i: (0, i, 0)),
        pl.BlockSpec((_BD, D), lambda i: (i, 0)),
        pl.BlockSpec((NW, _BD), lambda i: (0, i)),
        pl.BlockSpec((1, D), lambda i: (0, 0)),
        pl.BlockSpec((1, D), lambda i: (0, 0)),
    ],
    out_specs=pl.BlockSpec((_BD, D), lambda i: (i, 0)),
    out_shape=jax.ShapeDtypeStruct((N, D), jnp.float32),
)
def _finish_kernel(acc_ref, xw_ref, h_ref, b_ref, a_ref, o_ref):
    deg = jnp.sum(h_ref[...], axis=0) + 1.0
    dis = lax.rsqrt(deg)
    acc = acc_ref[...]
    t = (acc[0] + acc[1] + xw_ref[...]) * dis[:, None] + b_ref[...]
    o_ref[...] = jnp.where(t >= 0, t, a_ref[...] * t)


def kernel(x, edge_index, W, b, prelu_a):
    src = edge_index[0]
    dst = edge_index[1]
    src_p = jnp.concatenate(
        [src, jnp.full((EPAD - E,), N, dtype=jnp.int32)])
    dst_p = jnp.concatenate(
        [dst, N + jnp.arange(EPAD - E, dtype=jnp.int32) % (NPAD - N)])
    x_p = jnp.zeros((NPAD, D), x.dtype).at[:N].set(x)
    hist = _deg_kernel(dst_p)
    xw_s = _xw_kernel(x_p, W, hist)
    acc = _edge_kernel(xw_s, src_p, dst_p)
    out = _finish_kernel(acc, xw_s, hist, b.reshape(1, D),
                         prelu_a.reshape(1, D))
    return out


# speedup vs baseline: 1.4514x; 1.0327x over previous
import functools

import jax
import jax.numpy as jnp
from jax import lax
from jax.experimental import pallas as pl
from jax.experimental.pallas import tpu as pltpu
from jax.experimental.pallas import tpu_sc as plsc

N = 10000
E = 320000
D = 128
L = 16
NSC = 2
NTILE = 16
NW = NSC * NTILE
NPAD = 10240
CHUNK = 128
CPT0 = 100
CPT1 = 60
CPT = CPT0
EPT = (CPT0 + CPT1) * CHUNK // 2
EPAD = NTILE * (CPT0 + CPT1) * CHUNK
NACC = NPAD
APT = NACC // NTILE
ZCP = APT // CHUNK
NBUF = 2
NIDX = 4

_mesh = plsc.VectorSubcoreMesh(core_axis_name="c", subcore_axis_name="s",
                               num_cores=NSC, num_subcores=NTILE)


@functools.partial(
    pl.kernel,
    out_type=jax.ShapeDtypeStruct((NW, NPAD), jnp.float32),
    mesh=_mesh,
    scratch_types=[
        pltpu.VMEM((EPT,), jnp.int32),
        pltpu.VMEM((NPAD,), jnp.float32),
    ],
    compiler_params=pltpu.CompilerParams(needs_layout_passes=False),
)
def _deg_kernel(dst_hbm, out_hbm, dst_v, hist_v):
    c = lax.axis_index("c")
    s = lax.axis_index("s")
    wid = c * NTILE + s
    zeros16 = jnp.zeros((L,), jnp.float32)

    def zbody(i, _):
        hist_v[pl.ds(i * L, L)] = zeros16
        return ()

    lax.fori_loop(0, NPAD // L, zbody, (), unroll=8)
    pltpu.sync_copy(dst_hbm.at[pl.ds(wid * EPT, EPT)], dst_v)
    ones16 = jnp.ones((L,), jnp.float32)

    def body(k, _):
        idx = dst_v[pl.ds(k * L, L)]
        plsc.addupdate_scatter(hist_v, [idx], ones16)
        return ()

    lax.fori_loop(0, EPT // L, body, (), unroll=8)
    pltpu.sync_copy(hist_v, out_hbm.at[wid])


_BM = 1024


@functools.partial(
    pl.pallas_call,
    grid=(NPAD // _BM,),
    in_specs=[
        pl.BlockSpec((_BM, D), lambda i: (i, 0)),
        pl.BlockSpec((D, D), lambda i: (0, 0)),
        pl.BlockSpec((NW, _BM), lambda i: (0, i)),
    ],
    out_specs=pl.BlockSpec((_BM, D), lambda i: (i, 0)),
    out_shape=jax.ShapeDtypeStruct((NPAD, D), jnp.float32),
)
def _xw_kernel(x_ref, w_ref, h_ref, o_ref):
    deg = jnp.sum(h_ref[...], axis=0) + 1.0
    dis = lax.rsqrt(deg)
    xw = jnp.dot(x_ref[...], w_ref[...], preferred_element_type=jnp.float32)
    o_ref[...] = xw * dis[:, None]


@functools.partial(
    pl.kernel,
    out_type=jax.ShapeDtypeStruct((NSC, NACC, D), jnp.float32),
    mesh=_mesh,
    scratch_types=[
        [pltpu.VMEM((CHUNK,), jnp.int32)] * NIDX,
        [pltpu.VMEM((CHUNK,), jnp.int32)] * NIDX,
        pltpu.VMEM((NBUF, CHUNK, D), jnp.float32),
        pltpu.VMEM_SHARED((NACC, D), jnp.float32),
    ] + [pltpu.SemaphoreType.DMA] * (2 * NBUF + 2 * NIDX),
)
def _edge_kernel(xw_hbm, src_hbm, dst_hbm, out_hbm, src_v, dst_v, rows_v,
                 acc_sh, *sems):
    sg = sems[:NBUF]
    ss = sems[NBUF:2 * NBUF]
    ssrc = sems[2 * NBUF:2 * NBUF + NIDX]
    sdst = sems[2 * NBUF + NIDX:]
    c = lax.axis_index("c")
    s = lax.axis_index("s")
    cpt_c = jnp.where(c == 0, CPT0, CPT1)
    base = (c * NTILE * CPT0 + s * cpt_c) * CHUNK
    zeros16 = jnp.zeros((L,), jnp.float32)

    def zrow(i, _):
        for j in range(D // L):
            rows_v[0, i, pl.ds(j * L, L)] = zeros16
        return ()

    lax.fori_loop(0, CHUNK, zrow, ())
    for i in range(ZCP):
        pltpu.sync_copy(rows_v.at[0],
                        acc_sh.at[pl.ds(s * APT + i * CHUNK, CHUNK)])
    plsc.subcore_barrier()

    def start_idx(g, q):
        off = base + g * CHUNK
        pltpu.async_copy(src_hbm.at[pl.ds(off, CHUNK)], src_v[q], ssrc[q])
        pltpu.async_copy(dst_hbm.at[pl.ds(off, CHUNK)], dst_v[q], sdst[q])

    def wait_idx(g, q):
        off = base + g * CHUNK
        pltpu.make_async_copy(src_hbm.at[pl.ds(off, CHUNK)], src_v[q],
                              ssrc[q]).wait()
        pltpu.make_async_copy(dst_hbm.at[pl.ds(off, CHUNK)], dst_v[q],
                              sdst[q]).wait()

    def start_gather(q, b):
        pltpu.async_copy(xw_hbm.at[src_v[q]], rows_v.at[b], sg[b])

    def wait_gather(q, b):
        pltpu.make_async_copy(xw_hbm.at[src_v[q]], rows_v.at[b],
                              sg[b]).wait()

    def start_scatter(q, b):
        pltpu.async_copy(rows_v.at[b], acc_sh.at[dst_v[q]], ss[b],
                         add=True)

    def wait_scatter(q, b):
        pltpu.make_async_copy(rows_v.at[b], acc_sh.at[dst_v[q]],
                              ss[b]).wait()

    def body(g, b, q, first, issue_gather, issue_idx):
        wait_gather(q, b)
        if not first:
            wait_scatter((q - 1) % NIDX, 1 - b)
        start_scatter(q, b)
        if issue_gather:
            wait_idx(g + 1, (q + 1) % NIDX)
            start_gather((q + 1) % NIDX, 1 - b)
        if issue_idx:
            start_idx(g + 3, (q + 3) % NIDX)

    for q in range(NIDX - 1):
        start_idx(q, q)
    wait_idx(0, 0)
    start_gather(0, 0)
    body(0, 0, 0, True, True, True)

    def quad(k, _):
        for j in range(NIDX):
            body(1 + 4 * k + j, (1 + j) % 2, (1 + j) % NIDX,
                 False, True, True)
        return ()

    lax.fori_loop(0, (cpt_c - 4) // NIDX, quad, ())
    body(cpt_c - 3, (CPT - 3) % 2, (CPT - 3) % NIDX, False, True, False)
    body(cpt_c - 2, (CPT - 2) % 2, (CPT - 2) % NIDX, False, True, False)
    body(cpt_c - 1, (CPT - 1) % 2, (CPT - 1) % NIDX, False, False, False)
    wait_scatter((CPT - 1) % NIDX, (CPT - 1) % 2)
    plsc.subcore_barrier()

    for i in range(ZCP):
        r0 = s * APT + i * CHUNK
        pltpu.sync_copy(acc_sh.at[pl.ds(r0, CHUNK)],
                        out_hbm.at[c, pl.ds(r0, CHUNK)])


_BD = 1024


@functools.partial(
    pl.pallas_call,
    grid=(pl.cdiv(N, _BD),),
    in_specs=[
        pl.BlockSpec((NSC, _BD, D), lambda i: (0, i, 0)),
        pl.BlockSpec((_BD, D), lambda i: (i, 0)),
        pl.BlockSpec((NW, _BD), lambda i: (0, i)),
        pl.BlockSpec((1, D), lambda i: (0, 0)),
        pl.BlockSpec((1, D), lambda i: (0, 0)),
    ],
    out_specs=pl.BlockSpec((_BD, D), lambda i: (i, 0)),
    out_shape=jax.ShapeDtypeStruct((N, D), jnp.float32),
)
def _finish_kernel(acc_ref, xw_ref, h_ref, b_ref, a_ref, o_ref):
    deg = jnp.sum(h_ref[...], axis=0) + 1.0
    dis = lax.rsqrt(deg)
    acc = acc_ref[...]
    t = (acc[0] + acc[1] + xw_ref[...]) * dis[:, None] + b_ref[...]
    o_ref[...] = jnp.where(t >= 0, t, a_ref[...] * t)


def kernel(x, edge_index, W, b, prelu_a):
    src = edge_index[0]
    dst = edge_index[1]
    src_p = jnp.concatenate(
        [src, jnp.full((EPAD - E,), N, dtype=jnp.int32)])
    dst_p = jnp.concatenate(
        [dst, N + jnp.arange(EPAD - E, dtype=jnp.int32) % (NPAD - N)])
    x_p = jnp.zeros((NPAD, D), x.dtype).at[:N].set(x)
    hist = _deg_kernel(dst_p)
    xw_s = _xw_kernel(x_p, W, hist)
    acc = _edge_kernel(xw_s, src_p, dst_p)
    out = _finish_kernel(acc, xw_s, hist, b.reshape(1, D),
                         prelu_a.reshape(1, D))
    return out


# speedup vs baseline: 1.4971x; 1.0315x over previous
import functools

import jax
import jax.numpy as jnp
from jax import lax
from jax.experimental import pallas as pl
from jax.experimental.pallas import tpu as pltpu
from jax.experimental.pallas import tpu_sc as plsc

N = 10000
E = 320000
D = 128
L = 16
NSC = 2
NTILE = 16
NW = NSC * NTILE
NPAD = 10240
CHUNK = 128
CPT0 = 120
CPT1 = 40
CPT = CPT0
EPT = (CPT0 + CPT1) * CHUNK // 2
EPAD = NTILE * (CPT0 + CPT1) * CHUNK
NACC = NPAD
APT = NACC // NTILE
ZCP = APT // CHUNK
NBUF = 2
NIDX = 4

_mesh = plsc.VectorSubcoreMesh(core_axis_name="c", subcore_axis_name="s",
                               num_cores=NSC, num_subcores=NTILE)


@functools.partial(
    pl.kernel,
    out_type=jax.ShapeDtypeStruct((NW, NPAD), jnp.float32),
    mesh=_mesh,
    scratch_types=[
        pltpu.VMEM((EPT,), jnp.int32),
        pltpu.VMEM((NPAD,), jnp.float32),
    ],
    compiler_params=pltpu.CompilerParams(needs_layout_passes=False),
)
def _deg_kernel(dst_hbm, out_hbm, dst_v, hist_v):
    c = lax.axis_index("c")
    s = lax.axis_index("s")
    wid = c * NTILE + s
    zeros16 = jnp.zeros((L,), jnp.float32)

    def zbody(i, _):
        hist_v[pl.ds(i * L, L)] = zeros16
        return ()

    lax.fori_loop(0, NPAD // L, zbody, (), unroll=8)
    pltpu.sync_copy(dst_hbm.at[pl.ds(wid * EPT, EPT)], dst_v)
    ones16 = jnp.ones((L,), jnp.float32)

    def body(k, _):
        idx = dst_v[pl.ds(k * L, L)]
        plsc.addupdate_scatter(hist_v, [idx], ones16)
        return ()

    lax.fori_loop(0, EPT // L, body, (), unroll=8)
    pltpu.sync_copy(hist_v, out_hbm.at[wid])


_BM = 1024


@functools.partial(
    pl.pallas_call,
    grid=(NPAD // _BM,),
    in_specs=[
        pl.BlockSpec((_BM, D), lambda i: (i, 0)),
        pl.BlockSpec((D, D), lambda i: (0, 0)),
        pl.BlockSpec((NW, _BM), lambda i: (0, i)),
    ],
    out_specs=pl.BlockSpec((_BM, D), lambda i: (i, 0)),
    out_shape=jax.ShapeDtypeStruct((NPAD, D), jnp.float32),
)
def _xw_kernel(x_ref, w_ref, h_ref, o_ref):
    deg = jnp.sum(h_ref[...], axis=0) + 1.0
    dis = lax.rsqrt(deg)
    xw = jnp.dot(x_ref[...], w_ref[...], preferred_element_type=jnp.float32)
    o_ref[...] = xw * dis[:, None]


@functools.partial(
    pl.kernel,
    out_type=jax.ShapeDtypeStruct((NSC, NACC, D), jnp.float32),
    mesh=_mesh,
    scratch_types=[
        [pltpu.VMEM((CHUNK,), jnp.int32)] * NIDX,
        [pltpu.VMEM((CHUNK,), jnp.int32)] * NIDX,
        pltpu.VMEM((NBUF, CHUNK, D), jnp.float32),
        pltpu.VMEM_SHARED((NACC, D), jnp.float32),
    ] + [pltpu.SemaphoreType.DMA] * (2 * NBUF + 2 * NIDX),
)
def _edge_kernel(xw_hbm, src_hbm, dst_hbm, out_hbm, src_v, dst_v, rows_v,
                 acc_sh, *sems):
    sg = sems[:NBUF]
    ss = sems[NBUF:2 * NBUF]
    ssrc = sems[2 * NBUF:2 * NBUF + NIDX]
    sdst = sems[2 * NBUF + NIDX:]
    c = lax.axis_index("c")
    s = lax.axis_index("s")
    cpt_c = jnp.where(c == 0, CPT0, CPT1)
    base = (c * NTILE * CPT0 + s * cpt_c) * CHUNK
    zeros16 = jnp.zeros((L,), jnp.float32)

    def zrow(i, _):
        for j in range(D // L):
            rows_v[0, i, pl.ds(j * L, L)] = zeros16
        return ()

    lax.fori_loop(0, CHUNK, zrow, ())
    for i in range(ZCP):
        pltpu.sync_copy(rows_v.at[0],
                        acc_sh.at[pl.ds(s * APT + i * CHUNK, CHUNK)])
    plsc.subcore_barrier()

    def start_idx(g, q):
        off = base + g * CHUNK
        pltpu.async_copy(src_hbm.at[pl.ds(off, CHUNK)], src_v[q], ssrc[q])
        pltpu.async_copy(dst_hbm.at[pl.ds(off, CHUNK)], dst_v[q], sdst[q])

    def wait_idx(g, q):
        off = base + g * CHUNK
        pltpu.make_async_copy(src_hbm.at[pl.ds(off, CHUNK)], src_v[q],
                              ssrc[q]).wait()
        pltpu.make_async_copy(dst_hbm.at[pl.ds(off, CHUNK)], dst_v[q],
                              sdst[q]).wait()

    def start_gather(q, b):
        pltpu.async_copy(xw_hbm.at[src_v[q]], rows_v.at[b], sg[b])

    def wait_gather(q, b):
        pltpu.make_async_copy(xw_hbm.at[src_v[q]], rows_v.at[b],
                              sg[b]).wait()

    def start_scatter(q, b):
        pltpu.async_copy(rows_v.at[b], acc_sh.at[dst_v[q]], ss[b],
                         add=True)

    def wait_scatter(q, b):
        pltpu.make_async_copy(rows_v.at[b], acc_sh.at[dst_v[q]],
                              ss[b]).wait()

    def body(g, b, q, first, issue_gather, issue_idx):
        wait_gather(q, b)
        if not first:
            wait_scatter((q - 1) % NIDX, 1 - b)
        start_scatter(q, b)
        if issue_gather:
            wait_idx(g + 1, (q + 1) % NIDX)
            start_gather((q + 1) % NIDX, 1 - b)
        if issue_idx:
            start_idx(g + 3, (q + 3) % NIDX)

    for q in range(NIDX - 1):
        start_idx(q, q)
    wait_idx(0, 0)
    start_gather(0, 0)
    body(0, 0, 0, True, True, True)

    def quad(k, _):
        for j in range(NIDX):
            body(1 + 4 * k + j, (1 + j) % 2, (1 + j) % NIDX,
                 False, True, True)
        return ()

    lax.fori_loop(0, (cpt_c - 4) // NIDX, quad, ())
    body(cpt_c - 3, (CPT - 3) % 2, (CPT - 3) % NIDX, False, True, False)
    body(cpt_c - 2, (CPT - 2) % 2, (CPT - 2) % NIDX, False, True, False)
    body(cpt_c - 1, (CPT - 1) % 2, (CPT - 1) % NIDX, False, False, False)
    wait_scatter((CPT - 1) % NIDX, (CPT - 1) % 2)
    plsc.subcore_barrier()

    for i in range(ZCP):
        r0 = s * APT + i * CHUNK
        pltpu.sync_copy(acc_sh.at[pl.ds(r0, CHUNK)],
                        out_hbm.at[c, pl.ds(r0, CHUNK)])


_BD = 1024


@functools.partial(
    pl.pallas_call,
    grid=(pl.cdiv(N, _BD),),
    in_specs=[
        pl.BlockSpec((NSC, _BD, D), lambda i: (0, i, 0)),
        pl.BlockSpec((_BD, D), lambda i: (i, 0)),
        pl.BlockSpec((NW, _BD), lambda i: (0, i)),
        pl.BlockSpec((1, D), lambda i: (0, 0)),
        pl.BlockSpec((1, D), lambda i: (0, 0)),
    ],
    out_specs=pl.BlockSpec((_BD, D), lambda i: (i, 0)),
    out_shape=jax.ShapeDtypeStruct((N, D), jnp.float32),
)
def _finish_kernel(acc_ref, xw_ref, h_ref, b_ref, a_ref, o_ref):
    deg = jnp.sum(h_ref[...], axis=0) + 1.0
    dis = lax.rsqrt(deg)
    acc = acc_ref[...]
    t = (acc[0] + acc[1] + xw_ref[...]) * dis[:, None] + b_ref[...]
    o_ref[...] = jnp.where(t >= 0, t, a_ref[...] * t)


def kernel(x, edge_index, W, b, prelu_a):
    src = edge_index[0]
    dst = edge_index[1]
    src_p = jnp.concatenate(
        [src, jnp.full((EPAD - E,), N, dtype=jnp.int32)])
    dst_p = jnp.concatenate(
        [dst, N + jnp.arange(EPAD - E, dtype=jnp.int32) % (NPAD - N)])
    x_p = jnp.zeros((NPAD, D), x.dtype).at[:N].set(x)
    hist = _deg_kernel(dst_p)
    xw_s = _xw_kernel(x_p, W, hist)
    acc = _edge_kernel(xw_s, src_p, dst_p)
    out = _finish_kernel(acc, xw_s, hist, b.reshape(1, D),
                         prelu_a.reshape(1, D))
    return out


# speedup vs baseline: 1.5115x; 1.0096x over previous
import functools

import jax
import jax.numpy as jnp
from jax import lax
from jax.experimental import pallas as pl
from jax.experimental.pallas import tpu as pltpu
from jax.experimental.pallas import tpu_sc as plsc

N = 10000
E = 320000
D = 128
L = 16
NSC = 2
NTILE = 16
NW = NSC * NTILE
NPAD = 10240
CHUNK = 128
CPT0 = 136
CPT1 = 24
CPT = CPT0
EPT = (CPT0 + CPT1) * CHUNK // 2
EPAD = NTILE * (CPT0 + CPT1) * CHUNK
NACC = NPAD
APT = NACC // NTILE
ZCP = APT // CHUNK
NBUF = 2
NIDX = 4

_mesh = plsc.VectorSubcoreMesh(core_axis_name="c", subcore_axis_name="s",
                               num_cores=NSC, num_subcores=NTILE)


@functools.partial(
    pl.kernel,
    out_type=jax.ShapeDtypeStruct((NW, NPAD), jnp.float32),
    mesh=_mesh,
    scratch_types=[
        pltpu.VMEM((EPT,), jnp.int32),
        pltpu.VMEM((NPAD,), jnp.float32),
    ],
    compiler_params=pltpu.CompilerParams(needs_layout_passes=False),
)
def _deg_kernel(dst_hbm, out_hbm, dst_v, hist_v):
    c = lax.axis_index("c")
    s = lax.axis_index("s")
    wid = c * NTILE + s
    zeros16 = jnp.zeros((L,), jnp.float32)

    def zbody(i, _):
        hist_v[pl.ds(i * L, L)] = zeros16
        return ()

    lax.fori_loop(0, NPAD // L, zbody, (), unroll=8)
    pltpu.sync_copy(dst_hbm.at[pl.ds(wid * EPT, EPT)], dst_v)
    ones16 = jnp.ones((L,), jnp.float32)

    def body(k, _):
        idx = dst_v[pl.ds(k * L, L)]
        plsc.addupdate_scatter(hist_v, [idx], ones16)
        return ()

    lax.fori_loop(0, EPT // L, body, (), unroll=8)
    pltpu.sync_copy(hist_v, out_hbm.at[wid])


_BM = 1024


@functools.partial(
    pl.pallas_call,
    grid=(NPAD // _BM,),
    in_specs=[
        pl.BlockSpec((_BM, D), lambda i: (i, 0)),
        pl.BlockSpec((D, D), lambda i: (0, 0)),
        pl.BlockSpec((NW, _BM), lambda i: (0, i)),
    ],
    out_specs=pl.BlockSpec((_BM, D), lambda i: (i, 0)),
    out_shape=jax.ShapeDtypeStruct((NPAD, D), jnp.float32),
)
def _xw_kernel(x_ref, w_ref, h_ref, o_ref):
    deg = jnp.sum(h_ref[...], axis=0) + 1.0
    dis = lax.rsqrt(deg)
    xw = jnp.dot(x_ref[...], w_ref[...], preferred_element_type=jnp.float32)
    o_ref[...] = xw * dis[:, None]


@functools.partial(
    pl.kernel,
    out_type=jax.ShapeDtypeStruct((NSC, NACC, D), jnp.float32),
    mesh=_mesh,
    scratch_types=[
        [pltpu.VMEM((CHUNK,), jnp.int32)] * NIDX,
        [pltpu.VMEM((CHUNK,), jnp.int32)] * NIDX,
        pltpu.VMEM((NBUF, CHUNK, D), jnp.float32),
        pltpu.VMEM_SHARED((NACC, D), jnp.float32),
    ] + [pltpu.SemaphoreType.DMA] * (2 * NBUF + 2 * NIDX),
)
def _edge_kernel(xw_hbm, src_hbm, dst_hbm, out_hbm, src_v, dst_v, rows_v,
                 acc_sh, *sems):
    sg = sems[:NBUF]
    ss = sems[NBUF:2 * NBUF]
    ssrc = sems[2 * NBUF:2 * NBUF + NIDX]
    sdst = sems[2 * NBUF + NIDX:]
    c = lax.axis_index("c")
    s = lax.axis_index("s")
    cpt_c = jnp.where(c == 0, CPT0, CPT1)
    base = (c * NTILE * CPT0 + s * cpt_c) * CHUNK
    zeros16 = jnp.zeros((L,), jnp.float32)

    def zrow(i, _):
        for j in range(D // L):
            rows_v[0, i, pl.ds(j * L, L)] = zeros16
        return ()

    lax.fori_loop(0, CHUNK, zrow, ())
    for i in range(ZCP):
        pltpu.sync_copy(rows_v.at[0],
                        acc_sh.at[pl.ds(s * APT + i * CHUNK, CHUNK)])
    plsc.subcore_barrier()

    def start_idx(g, q):
        off = base + g * CHUNK
        pltpu.async_copy(src_hbm.at[pl.ds(off, CHUNK)], src_v[q], ssrc[q])
        pltpu.async_copy(dst_hbm.at[pl.ds(off, CHUNK)], dst_v[q], sdst[q])

    def wait_idx(g, q):
        off = base + g * CHUNK
        pltpu.make_async_copy(src_hbm.at[pl.ds(off, CHUNK)], src_v[q],
                              ssrc[q]).wait()
        pltpu.make_async_copy(dst_hbm.at[pl.ds(off, CHUNK)], dst_v[q],
                              sdst[q]).wait()

    def start_gather(q, b):
        pltpu.async_copy(xw_hbm.at[src_v[q]], rows_v.at[b], sg[b])

    def wait_gather(q, b):
        pltpu.make_async_copy(xw_hbm.at[src_v[q]], rows_v.at[b],
                              sg[b]).wait()

    def start_scatter(q, b):
        pltpu.async_copy(rows_v.at[b], acc_sh.at[dst_v[q]], ss[b],
                         add=True)

    def wait_scatter(q, b):
        pltpu.make_async_copy(rows_v.at[b], acc_sh.at[dst_v[q]],
                              ss[b]).wait()

    def body(g, b, q, first, issue_gather, issue_idx):
        wait_gather(q, b)
        if not first:
            wait_scatter((q - 1) % NIDX, 1 - b)
        start_scatter(q, b)
        if issue_gather:
            wait_idx(g + 1, (q + 1) % NIDX)
            start_gather((q + 1) % NIDX, 1 - b)
        if issue_idx:
            start_idx(g + 3, (q + 3) % NIDX)

    for q in range(NIDX - 1):
        start_idx(q, q)
    wait_idx(0, 0)
    start_gather(0, 0)
    body(0, 0, 0, True, True, True)

    def quad(k, _):
        for j in range(NIDX):
            body(1 + 4 * k + j, (1 + j) % 2, (1 + j) % NIDX,
                 False, True, True)
        return ()

    lax.fori_loop(0, (cpt_c - 4) // NIDX, quad, ())
    body(cpt_c - 3, (CPT - 3) % 2, (CPT - 3) % NIDX, False, True, False)
    body(cpt_c - 2, (CPT - 2) % 2, (CPT - 2) % NIDX, False, True, False)
    body(cpt_c - 1, (CPT - 1) % 2, (CPT - 1) % NIDX, False, False, False)
    wait_scatter((CPT - 1) % NIDX, (CPT - 1) % 2)
    plsc.subcore_barrier()

    for i in range(ZCP):
        r0 = s * APT + i * CHUNK
        pltpu.sync_copy(acc_sh.at[pl.ds(r0, CHUNK)],
                        out_hbm.at[c, pl.ds(r0, CHUNK)])


_BD = 1024


@functools.partial(
    pl.pallas_call,
    grid=(pl.cdiv(N, _BD),),
    in_specs=[
        pl.BlockSpec((NSC, _BD, D), lambda i: (0, i, 0)),
        pl.BlockSpec((_BD, D), lambda i: (i, 0)),
        pl.BlockSpec((NW, _BD), lambda i: (0, i)),
        pl.BlockSpec((1, D), lambda i: (0, 0)),
        pl.BlockSpec((1, D), lambda i: (0, 0)),
    ],
    out_specs=pl.BlockSpec((_BD, D), lambda i: (i, 0)),
    out_shape=jax.ShapeDtypeStruct((N, D), jnp.float32),
)
def _finish_kernel(acc_ref, xw_ref, h_ref, b_ref, a_ref, o_ref):
    deg = jnp.sum(h_ref[...], axis=0) + 1.0
    dis = lax.rsqrt(deg)
    acc = acc_ref[...]
    t = (acc[0] + acc[1] + xw_ref[...]) * dis[:, None] + b_ref[...]
    o_ref[...] = jnp.where(t >= 0, t, a_ref[...] * t)


def kernel(x, edge_index, W, b, prelu_a):
    src = edge_index[0]
    dst = edge_index[1]
    src_p = jnp.concatenate(
        [src, jnp.full((EPAD - E,), N, dtype=jnp.int32)])
    dst_p = jnp.concatenate(
        [dst, N + jnp.arange(EPAD - E, dtype=jnp.int32) % (NPAD - N)])
    x_p = jnp.zeros((NPAD, D), x.dtype).at[:N].set(x)
    hist = _deg_kernel(dst_p)
    xw_s = _xw_kernel(x_p, W, hist)
    acc = _edge_kernel(xw_s, src_p, dst_p)
    out = _finish_kernel(acc, xw_s, hist, b.reshape(1, D),
                         prelu_a.reshape(1, D))
    return out
